# baseline (speedup 1.0000x reference)
import jax
import jax.numpy as jnp
from jax import lax
from jax.experimental import pallas as pl
from jax.experimental.pallas import tpu as pltpu
from jax.experimental.pallas import tpu_sc as plsc

IMG = 8192
N = 24576
NC = 2
NS = 16
L = 16
NW = NC * NS
PER = N // NW
CH = 128
NCH = PER // CH
NV = PER // L


def _body(img_hbm, len_hbm, pix_hbm, out_hbm,
          pair_v, len_v, idx_v, val_v, acc_v, all_v,
          sem_p, sem_len, *gsems):
    cid = lax.axis_index("c")
    sid = lax.axis_index("s")
    wid = cid * NS + sid
    base = wid * PER

    cp_p = pltpu.async_copy(pix_hbm.at[pl.ds(2 * base, 2 * PER)], pair_v, sem_p)
    cp_len = pltpu.async_copy(len_hbm.at[pl.ds(base, PER)], len_v, sem_len)
    cp_p.wait()

    lane = lax.iota(jnp.int32, L)
    qr = (lane * 2) & (L - 1)
    qc = qr + 1
    lo = lane < (L // 2)

    gathers = []
    for jj in range(NCH):
        for k in range(CH // L):
            e0 = jj * CH + k * L
            a = pair_v[pl.ds(2 * e0, L)]
            b = pair_v[pl.ds(2 * e0 + L, L)]
            rr = jnp.where(lo, a.at[qr].get(mode="promise_in_bounds"),
                           b.at[qr].get(mode="promise_in_bounds"))
            cc = jnp.where(lo, a.at[qc].get(mode="promise_in_bounds"),
                           b.at[qc].get(mode="promise_in_bounds"))
            addr = ((rr >> 3) << 16) + ((cc >> 7) << 10) + ((rr & 7) << 7) + (cc & 127)
            idx_v[jj, pl.ds(k * L, L)] = addr
        gathers.append(
            pltpu.async_copy(img_hbm.at[idx_v.at[jj]], val_v.at[jj], gsems[jj]))

    cp_len.wait()

    acc = jnp.zeros((L,), jnp.float32)
    for jj in range(NCH):
        gathers[jj].wait()
        for k in range(CH // L):
            e0 = jj * CH + k * L
            acc = acc + val_v[jj, pl.ds(k * L, L)] * len_v[pl.ds(e0, L)]
    acc_v[0, :] = acc

    pltpu.sync_copy(acc_v, out_hbm.at[pl.ds(wid, 1)])
    plsc.subcore_barrier()

    @pl.when(sid == 0)
    def _():
        pltpu.sync_copy(out_hbm.at[pl.ds(cid * NS, NS)], all_v)
        tot = jnp.zeros((L,), jnp.float32)
        for i in range(NS):
            tot = tot + all_v[i, :]
        lane = lax.iota(jnp.int32, L)
        for sh in (1, 2, 4, 8):
            tot = tot + tot.at[lane ^ sh].get(mode="promise_in_bounds")
        acc_v[0, :] = tot
        pltpu.sync_copy(acc_v, out_hbm.at[pl.ds(cid * NS, 1)])


def kernel(img, lengths, idx):
    idx = idx.astype(jnp.int32)
    img_flat = (
        img.reshape(IMG // 8, 8, IMG // 128, 128)
        .transpose(0, 2, 1, 3)
        .reshape(-1)
    )
    mesh = plsc.VectorSubcoreMesh(core_axis_name="c", subcore_axis_name="s")
    out = pl.kernel(
        _body,
        mesh=mesh,
        out_type=jax.ShapeDtypeStruct((NW, L), jnp.float32),
        scratch_types=[
            pltpu.VMEM((2 * PER,), jnp.int32),
            pltpu.VMEM((PER,), jnp.float32),
            pltpu.VMEM((NCH, CH), jnp.int32),
            pltpu.VMEM((NCH, CH), jnp.float32),
            pltpu.VMEM((1, L), jnp.float32),
            pltpu.VMEM((NS, L), jnp.float32),
        ] + [pltpu.SemaphoreType.DMA] * (2 + NCH),
    )(img_flat, lengths, idx.reshape(-1))
    return out[0, 0] + out[NS, 0]


# speedup vs baseline: 1.6192x; 1.6192x over previous
import jax
import jax.numpy as jnp
from jax import lax
from jax.experimental import pallas as pl
from jax.experimental.pallas import tpu as pltpu
from jax.experimental.pallas import tpu_sc as plsc

IMG = 8192
N = 24576
NC = 2
NS = 16
L = 16
NW = NC * NS
PER = N // NW
CH = 128
NCH = PER // CH
NV = PER // L


def _body(img_hbm, len_hbm, addr_hbm, out_hbm,
          len_v, idx_v, val_v, acc_v, all_v,
          sem_len, *sems):
    isem = sems[:NCH]
    gsem = sems[NCH:]
    cid = lax.axis_index("c")
    sid = lax.axis_index("s")
    wid = cid * NS + sid
    base = wid * PER

    cp_len = pltpu.async_copy(len_hbm.at[pl.ds(base, PER)], len_v, sem_len)
    cp_idx = [
        pltpu.async_copy(addr_hbm.at[pl.ds(base + jj * CH, CH)],
                         idx_v.at[jj], isem[jj])
        for jj in range(NCH)
    ]
    gathers = []
    for jj in range(NCH):
        cp_idx[jj].wait()
        gathers.append(
            pltpu.async_copy(img_hbm.at[idx_v.at[jj]], val_v.at[jj], gsem[jj]))

    cp_len.wait()

    acc = jnp.zeros((L,), jnp.float32)
    for jj in range(NCH):
        gathers[jj].wait()
        for k in range(CH // L):
            e0 = jj * CH + k * L
            acc = acc + val_v[jj, pl.ds(k * L, L)] * len_v[pl.ds(e0, L)]
    acc_v[0, :] = acc

    pltpu.sync_copy(acc_v, out_hbm.at[pl.ds(wid, 1)])
    plsc.subcore_barrier()

    @pl.when(sid == 0)
    def _():
        pltpu.sync_copy(out_hbm.at[pl.ds(cid * NS, NS)], all_v)
        tot = jnp.zeros((L,), jnp.float32)
        for i in range(NS):
            tot = tot + all_v[i, :]
        lane = lax.iota(jnp.int32, L)
        for sh in (1, 2, 4, 8):
            tot = tot + tot.at[lane ^ sh].get(mode="promise_in_bounds")
        acc_v[0, :] = tot
        pltpu.sync_copy(acc_v, out_hbm.at[pl.ds(cid * NS, 1)])


def kernel(img, lengths, idx):
    idx = idx.astype(jnp.int32)
    rr = idx[:, 0]
    cc = idx[:, 1]
    addr = ((rr >> 3) << 16) + ((cc >> 7) << 10) + ((rr & 7) << 7) + (cc & 127)
    img_flat = (
        img.reshape(IMG // 8, 8, IMG // 128, 128)
        .transpose(0, 2, 1, 3)
        .reshape(-1)
    )
    mesh = plsc.VectorSubcoreMesh(core_axis_name="c", subcore_axis_name="s")
    out = pl.kernel(
        _body,
        mesh=mesh,
        out_type=jax.ShapeDtypeStruct((NW, L), jnp.float32),
        scratch_types=[
            pltpu.VMEM((PER,), jnp.float32),
            pltpu.VMEM((NCH, CH), jnp.int32),
            pltpu.VMEM((NCH, CH), jnp.float32),
            pltpu.VMEM((1, L), jnp.float32),
            pltpu.VMEM((NS, L), jnp.float32),
        ] + [pltpu.SemaphoreType.DMA] * (1 + 2 * NCH),
    )(img_flat, lengths, addr)
    return out[0, 0] + out[NS, 0]


# speedup vs baseline: 1.7948x; 1.1085x over previous
import jax
import jax.numpy as jnp
from jax import lax
from jax.experimental import pallas as pl
from jax.experimental.pallas import tpu as pltpu
from jax.experimental.pallas import tpu_sc as plsc

IMG = 8192
N = 24576
NC = 2
NS = 16
L = 16
NW = NC * NS
PER = N // NW
CH = 128
NCH = PER // CH
NV = PER // L


def _body(img_hbm, len_hbm, addr_hbm, out_hbm,
          len_v, idx_v, val_v, acc_v,
          sem_len, *sems):
    isem = sems[:NCH]
    gsem = sems[NCH:]
    cid = lax.axis_index("c")
    sid = lax.axis_index("s")
    wid = cid * NS + sid
    base = wid * PER

    cp_len = pltpu.async_copy(len_hbm.at[pl.ds(base, PER)], len_v, sem_len)
    cp_idx = [
        pltpu.async_copy(addr_hbm.at[pl.ds(base + jj * CH, CH)],
                         idx_v.at[jj], isem[jj])
        for jj in range(NCH)
    ]
    gathers = []
    for jj in range(NCH):
        cp_idx[jj].wait()
        gathers.append(
            pltpu.async_copy(img_hbm.at[idx_v.at[jj]], val_v.at[jj], gsem[jj]))

    cp_len.wait()

    acc = jnp.zeros((L,), jnp.float32)
    for jj in range(NCH):
        gathers[jj].wait()
        for k in range(CH // L):
            e0 = jj * CH + k * L
            acc = acc + val_v[jj, pl.ds(k * L, L)] * len_v[pl.ds(e0, L)]

    lane = lax.iota(jnp.int32, L)
    for sh in (1, 2, 4, 8):
        acc = acc + acc.at[lane ^ sh].get(mode="promise_in_bounds")
    acc_v[0, :] = acc
    pltpu.sync_copy(acc_v, out_hbm.at[pl.ds(wid, 1)])


def kernel(img, lengths, idx):
    idx = idx.astype(jnp.int32)
    rr = idx[:, 0]
    cc = idx[:, 1]
    addr = ((rr >> 3) << 16) + ((cc >> 7) << 10) + ((rr & 7) << 7) + (cc & 127)
    img_flat = (
        img.reshape(IMG // 8, 8, IMG // 128, 128)
        .transpose(0, 2, 1, 3)
        .reshape(-1)
    )
    mesh = plsc.VectorSubcoreMesh(core_axis_name="c", subcore_axis_name="s")
    out = pl.kernel(
        _body,
        mesh=mesh,
        out_type=jax.ShapeDtypeStruct((NW, L), jnp.float32),
        scratch_types=[
            pltpu.VMEM((PER,), jnp.float32),
            pltpu.VMEM((NCH, CH), jnp.int32),
            pltpu.VMEM((NCH, CH), jnp.float32),
            pltpu.VMEM((1, L), jnp.float32),
        ] + [pltpu.SemaphoreType.DMA] * (1 + 2 * NCH),
    )(img_flat, lengths, addr)
    return jnp.sum(out[:, 0])
